# Initial kernel scaffold; baseline (speedup 1.0000x reference)
import jax
import jax.numpy as jnp
from jax.experimental import pallas as pl


def kernel(x, edge_index, W0, b0, W1, b1, W2, b2):
    raise NotImplementedError("write your pallas kernel here")


# speedup vs baseline: 2.3389x; 2.3389x over previous
import jax
import jax.numpy as jnp
from jax import lax
from jax.experimental import pallas as pl
from jax.experimental.pallas import tpu as pltpu
from jax.experimental.pallas import tpu_sc as plsc

N = 10000
D = 128
E = 320000

NC = 2
NS = 16
NW = NC * NS

N_PAD = 10112
TRASH = 10048
APT = N_PAD // NS

K = 128
KW = K // 2
NCH = 80
EPW = NCH * K
E_PAD = NW * EPW

HALF = N_PAD // 2
HH = 5120
HPT = HH // NS

BR = 2528
NBLK = N_PAD // BR

_mesh = plsc.VectorSubcoreMesh(core_axis_name="c", subcore_axis_name="s")
_sc_params = pltpu.CompilerParams(needs_layout_passes=False)


def _unpack_row(wrow, irow, sub, clamp):
    for q in range(K // 32):
        v = wrow[pl.ds(q * 16, 16)]
        lo = (v & 0xFFFF) - sub
        hi = lax.shift_right_logical(v, 16) - sub
        if clamp is not None:
            lo = jnp.where((lo >= 0) & (lo < clamp), lo, clamp)
            hi = jnp.where((hi >= 0) & (hi < clamp), hi, clamp)
        irow[pl.ds(q * 32, 16)] = lo
        irow[pl.ds(q * 32 + 16, 16)] = hi


def _fill16(buf, n, value):
    @pl.loop(0, n)
    def _(r):
        buf[r, :] = value


def _deg_body(srcw_hbm, dstw_hbm, hist_hbm, swr, dwr, sidx, didx, ones_o_v,
              ones_i_v, hist_sh, sem_i):
    c = lax.axis_index("c")
    s = lax.axis_index("s")
    w = s * NC + c

    def words_start(j, b):
        pltpu.async_copy(srcw_hbm.at[w, j], swr.at[b], sem_i)
        pltpu.async_copy(dstw_hbm.at[w, j], dwr.at[b], sem_i)

    def words_wait(j, b):
        pltpu.make_async_copy(srcw_hbm.at[w, j], swr.at[b], sem_i).wait()
        pltpu.make_async_copy(dstw_hbm.at[w, j], dwr.at[b], sem_i).wait()

    lane = lax.iota(jnp.int32, 16)
    row_o = jnp.where(lane < 8, 1.0, 0.0).astype(jnp.float32)
    row_i = jnp.where(lane < 8, 0.0, 1.0).astype(jnp.float32)
    zeros16 = jnp.zeros((16,), jnp.float32)

    base = s * HPT
    for p in range(2):
        _fill16(ones_o_v, K, zeros16)
        for j in range(HPT // K):
            pltpu.sync_copy(ones_o_v, hist_sh.at[pl.ds(base + j * K, K)])
        pltpu.sync_copy(ones_o_v.at[pl.ds(0, HPT % K)],
                        hist_sh.at[pl.ds(base + (HPT // K) * K, HPT % K)])
        _fill16(ones_o_v, K, row_o)
        _fill16(ones_i_v, K, row_i)
        words_start(0, 0)
        plsc.subcore_barrier()
        words_wait(0, 0)

        @pl.loop(0, NCH, step=2)
        def _(j0):
            for b in range(2):
                jn = j0 + b + 1

                @pl.when(jn < NCH)
                def _():
                    words_start(jn, 1 - b)

                _unpack_row(swr.at[b], sidx.at[0], p * HALF, HALF)
                _unpack_row(dwr.at[b], didx.at[0], p * HALF, HALF)
                pltpu.sync_copy(ones_o_v, hist_sh.at[sidx.at[0]], add=True)
                pltpu.sync_copy(ones_i_v, hist_sh.at[didx.at[0]], add=True)

                @pl.when(jn < NCH)
                def _():
                    words_wait(jn, 1 - b)

        plsc.subcore_barrier()
        pltpu.sync_copy(hist_sh.at[pl.ds(base, HPT)],
                        hist_hbm.at[c, p, pl.ds(base, HPT)])
        if p == 0:
            plsc.subcore_barrier()


_deg_call = pl.kernel(
    _deg_body,
    out_type=jax.ShapeDtypeStruct((NC, 2, HH, 16), jnp.float32),
    mesh=_mesh,
    compiler_params=_sc_params,
    scratch_types=[
        pltpu.VMEM((2, KW), jnp.int32),
        pltpu.VMEM((2, KW), jnp.int32),
        pltpu.VMEM((1, K), jnp.int32),
        pltpu.VMEM((1, K), jnp.int32),
        pltpu.VMEM((K, 16), jnp.float32),
        pltpu.VMEM((K, 16), jnp.float32),
        pltpu.VMEM_SHARED((HH, 16), jnp.float32),
        pltpu.SemaphoreType.DMA,
    ],
)


def _prop_body(h_hbm, srcw_hbm, dstw_hbm, out_hbm, swr, dwr, sir, dir_, rows,
               agg_sh, sem_i, sem_g):
    c = lax.axis_index("c")
    s = lax.axis_index("s")
    w = s * NC + c

    def words_start(j, b):
        pltpu.async_copy(srcw_hbm.at[w, j], swr.at[b], sem_i)
        pltpu.async_copy(dstw_hbm.at[w, j], dwr.at[b], sem_i)

    def words_wait(j, b):
        pltpu.make_async_copy(srcw_hbm.at[w, j], swr.at[b], sem_i).wait()
        pltpu.make_async_copy(dstw_hbm.at[w, j], dwr.at[b], sem_i).wait()

    def unpack(b):
        _unpack_row(swr.at[b], sir.at[b], 0, None)
        _unpack_row(dwr.at[b], dir_.at[b], 0, None)

    def gather_start(b):
        pltpu.async_copy(h_hbm.at[sir.at[b]], rows.at[b], sem_g)

    def gather_wait(b):
        pltpu.make_async_copy(h_hbm.at[sir.at[b]], rows.at[b], sem_g).wait()

    def scatter(b):
        pltpu.sync_copy(rows.at[b], agg_sh.at[dir_.at[b]], add=True)

    words_start(0, 0)

    zeros16 = jnp.zeros((16,), jnp.float32)

    @pl.loop(0, K)
    def _(r):
        for j in range(D // 16):
            rows[0, r, pl.ds(j * 16, 16)] = zeros16

    zbase = s * APT
    for j in range(APT // K):
        pltpu.sync_copy(rows.at[0], agg_sh.at[pl.ds(zbase + j * K, K)])
    pltpu.sync_copy(rows.at[0, pl.ds(0, APT % K)],
                    agg_sh.at[pl.ds(zbase + (APT // K) * K, APT % K)])

    words_wait(0, 0)
    words_start(1, 1)
    unpack(0)
    plsc.subcore_barrier()
    gather_start(0)

    @pl.loop(0, NCH, step=2)
    def _(j0):
        words_wait(j0 + 1, 1)
        unpack(1)
        gather_wait(0)
        gather_start(1)

        @pl.when(j0 + 2 < NCH)
        def _():
            words_start(j0 + 2, 0)

        scatter(0)

        @pl.when(j0 + 2 < NCH)
        def _():
            words_wait(j0 + 2, 0)
            unpack(0)

        gather_wait(1)

        @pl.when(j0 + 2 < NCH)
        def _():
            gather_start(0)

        @pl.when(j0 + 3 < NCH)
        def _():
            words_start(j0 + 3, 1)

        scatter(1)

    plsc.subcore_barrier()
    pltpu.sync_copy(agg_sh.at[pl.ds(zbase, APT)],
                    out_hbm.at[c, pl.ds(zbase, APT)])


_prop_call = pl.kernel(
    _prop_body,
    out_type=jax.ShapeDtypeStruct((NC, N_PAD, D), jnp.float32),
    mesh=_mesh,
    compiler_params=_sc_params,
    scratch_types=[
        pltpu.VMEM((2, KW), jnp.int32),
        pltpu.VMEM((2, KW), jnp.int32),
        pltpu.VMEM((2, K), jnp.int32),
        pltpu.VMEM((2, K), jnp.int32),
        pltpu.VMEM((2, K, D), jnp.float32),
        pltpu.VMEM_SHARED((N_PAD, D), jnp.float32),
        pltpu.SemaphoreType.DMA,
        pltpu.SemaphoreType.DMA,
    ],
)


def _norm_body(hist_ref, no_ref, ni_ref):
    h = hist_ref[...]
    full = jnp.concatenate([h[:, 0, :HALF, :], h[:, 1, :HALF, :]], axis=1)
    deg_o = jnp.sum(full[:, :, :8], axis=(0, 2)) * 0.125
    deg_i = jnp.sum(full[:, :, 8:], axis=(0, 2)) * 0.125
    no_ref[...] = lax.rsqrt(jnp.maximum(deg_o, 1.0))
    ni_ref[...] = lax.rsqrt(jnp.maximum(deg_i, 1.0))


_norm_call = pl.pallas_call(
    _norm_body,
    out_shape=(
        jax.ShapeDtypeStruct((N_PAD,), jnp.float32),
        jax.ShapeDtypeStruct((N_PAD,), jnp.float32),
    ),
)


def _scale_body(x_ref, no_ref, out_ref):
    out_ref[...] = x_ref[...] * no_ref[...]


_scale_call = pl.pallas_call(
    _scale_body,
    grid=(NBLK,),
    in_specs=[
        pl.BlockSpec((BR, D), lambda i: (i, 0)),
        pl.BlockSpec((BR, 1), lambda i: (i, 0)),
    ],
    out_specs=pl.BlockSpec((BR, D), lambda i: (i, 0)),
    out_shape=jax.ShapeDtypeStruct((N_PAD, D), jnp.float32),
)


def _layer_body(agg_ref, ni_ref, no_ref, w_ref, b_ref, f_ref, out_ref):
    t = (agg_ref[0] + agg_ref[1]) * ni_ref[...]
    z = lax.dot_general(t, w_ref[...], (((1,), (0,)), ((), ())),
                        precision=lax.Precision.HIGHEST,
                        preferred_element_type=jnp.float32)
    z = z + b_ref[...]
    zm = jnp.maximum(z, 0.0) * no_ref[...]
    out_ref[...] = jnp.where(f_ref[0, 0] > 0.5, z, zm)


_layer_call = pl.pallas_call(
    _layer_body,
    grid=(NBLK,),
    in_specs=[
        pl.BlockSpec((NC, BR, D), lambda i: (0, i, 0)),
        pl.BlockSpec((BR, 1), lambda i: (i, 0)),
        pl.BlockSpec((BR, 1), lambda i: (i, 0)),
        pl.BlockSpec((D, D), lambda i: (0, 0)),
        pl.BlockSpec((1, D), lambda i: (0, 0)),
        pl.BlockSpec((1, 1), lambda i: (0, 0)),
    ],
    out_specs=pl.BlockSpec((BR, D), lambda i: (i, 0)),
    out_shape=jax.ShapeDtypeStruct((N_PAD, D), jnp.float32),
)


@jax.jit
def kernel(x, edge_index, W0, b0, W1, b1, W2, b2):
    src = edge_index[0]
    dst = edge_index[1]

    def _pack(idx):
        p = jnp.full((E_PAD,), TRASH, jnp.int32).at[:E].set(idx)
        p = p.reshape(E_PAD // 2, 2)
        return (p[:, 0] | (p[:, 1] << 16)).reshape(NW, NCH, KW)

    src_m = _pack(src)
    dst_m = _pack(dst)
    x_pad = jnp.zeros((N_PAD, D), jnp.float32).at[:N].set(x)

    hist = _deg_call(src_m, dst_m)
    norm_out, norm_in = _norm_call(hist)
    no_col = norm_out.reshape(N_PAD, 1)
    ni_col = norm_in.reshape(N_PAD, 1)

    h0 = _scale_call(x_pad, no_col)

    Ws = jnp.stack([W0, W1, W2])
    bs = jnp.stack([b0.reshape(1, D), b1.reshape(1, D), b2.reshape(1, D)])
    fs = jnp.array([0.0, 0.0, 1.0], jnp.float32).reshape(3, 1, 1)

    def _layer_step(h, xs):
        Wl, bl, fl = xs
        agg = _prop_call(h, src_m, dst_m)
        return _layer_call(agg, ni_col, no_col, Wl, bl, fl), None

    h, _ = lax.scan(_layer_step, h0, (Ws, bs, fs))
    return h[:N]
